# baseline (speedup 1.0000x reference)
import functools
import math

import jax
import jax.numpy as jnp
from jax import lax
from jax.experimental import pallas as pl
from jax.experimental.pallas import tpu as pltpu
from jax.experimental.pallas import tpu_sc as plsc

SMOOTH = 0.1
CONF = 1.0 - SMOOTH

_NC = 2
_NS = 16
_NW = _NC * _NS
_LANES = 16

_GROUPS_PER_W = 2
_ROWS_SC = _NW * _GROUPS_PER_W * 8
_WC = 3200


def _sc_body(v, flat_ignored, x_hbm, tgt_hbm, m_hbm, out_hbm,
             tgt_v, m_v, buf0, buf1, acc_out, sem0, sem1):
    wid = lax.axis_index("s") * _NC + lax.axis_index("c")
    rpw = _GROUPS_PER_W * 8
    rbase = wid * rpw
    nchunks = v // _WC
    bufs = (buf0, buf1)
    sems = (sem0, sem1)

    pltpu.sync_copy(tgt_hbm.at[pl.ds(rbase, rpw)], tgt_v)
    pltpu.sync_copy(m_hbm.at[pl.ds(rbase, rpw)], m_v)
    tgt16 = tgt_v[pl.ds(0, _LANES)]
    m16 = m_v[pl.ds(0, _LANES)]

    ts16 = jnp.zeros((_LANES,), jnp.float32)
    g16 = jnp.zeros((_LANES,), jnp.float32)
    zero = jnp.zeros((_LANES,), jnp.float32)
    lane = lax.broadcasted_iota(jnp.int32, (_LANES,), 0)

    h = pltpu.async_copy(
        x_hbm.at[pl.ds(rbase, 8), pl.ds(0, _WC)], buf0, sem0)
    for gc in range(_GROUPS_PER_W * nchunks):
        g, c = divmod(gc, nchunks)
        if gc + 1 < _GROUPS_PER_W * nchunks:
            gn, cn = divmod(gc + 1, nchunks)
            h_next = pltpu.async_copy(
                x_hbm.at[pl.ds(rbase + gn * 8, 8), pl.ds(cn * _WC, _WC)],
                bufs[(gc + 1) % 2], sems[(gc + 1) % 2])
        h.wait()
        buf = bufs[gc % 2]

        for r in range(8):
            row_ref = buf.at[r]
            ri = g * 8 + r

            def _chunks(i, accs, row_ref=row_ref):
                return tuple(
                    a + row_ref[pl.ds((i * 8 + k) * _LANES, _LANES)]
                    for k, a in enumerate(accs))

            a0, a1, a2, a3, a4, a5, a6, a7 = lax.fori_loop(
                0, _WC // (8 * _LANES), _chunks, (zero,) * 8)
            acc = ((a0 + a1) + (a2 + a3)) + ((a4 + a5) + (a6 + a7))
            mrow = m16[ri]
            ts16 = ts16 + acc * mrow

            t_r = tgt16[ri]
            local = t_r - c * _WC
            off = pl.multiple_of(
                jnp.clip(local & ~(_LANES - 1), 0, _WC - _LANES), _LANES)
            gv16 = row_ref[pl.ds(off, _LANES)]
            eq = (lane + off) == local
            g16 = g16 + jnp.where(eq, gv16 * mrow, 0.0)

        if gc + 1 < _GROUPS_PER_W * nchunks:
            h = h_next

    acc_out[pl.ds(0, _LANES)] = ts16
    acc_out[pl.ds(_LANES, _LANES)] = g16
    acc_out[pl.ds(2 * _LANES, _LANES)] = m16
    pltpu.sync_copy(acc_out, out_hbm.at[wid])


def _make_sc(n_rows, v):
    mesh = plsc.VectorSubcoreMesh(core_axis_name="c", subcore_axis_name="s")
    return pl.kernel(
        functools.partial(_sc_body, v, None),
        out_type=jax.ShapeDtypeStruct((_NW, 3 * _LANES), jnp.float32),
        mesh=mesh,
        scratch_types=[
            pltpu.VMEM((_GROUPS_PER_W * 8,), jnp.int32),
            pltpu.VMEM((_GROUPS_PER_W * 8,), jnp.float32),
            pltpu.VMEM((8, _WC), jnp.float32),
            pltpu.VMEM((8, _WC), jnp.float32),
            pltpu.VMEM((3 * _LANES,), jnp.float32),
            pltpu.SemaphoreType.DMA,
            pltpu.SemaphoreType.DMA,
        ],
        compiler_params=pltpu.CompilerParams(use_tc_tiling_on_sc=True),
    )


def _tc_body(nsteps_i, eps, v,
             x_ref, m_ref, tgt_ref, out_ref, acc, acc_m):
    i = pl.program_id(0)

    @pl.when(i == 0)
    def _init():
        acc[0, 0] = 0.0
        acc_m[0, 0] = 0.0

    x = x_ref[...]
    m = m_ref[...]
    tgt = tgt_ref[...]
    br = x.shape[0]
    cols = lax.broadcasted_iota(jnp.int32, (br, v), 1)
    eq = cols == tgt
    rowsum = jnp.sum(x, axis=1, keepdims=True)
    grow = jnp.sum(jnp.where(eq, x, 0.0), axis=1, keepdims=True)
    acc[0, 0] += jnp.sum((-eps * rowsum + (eps - CONF) * grow) * m)
    acc_m[0, 0] += jnp.sum(m)

    @pl.when(i == nsteps_i - 1)
    def _fin():
        out_ref[0, 0] = acc[0, 0]
        out_ref[0, 1] = acc_m[0, 0]


def _make_tc(n_rows, v, block_r, row_off):
    ni = (n_rows - row_off) // block_r
    off_blocks = row_off // block_r
    eps = SMOOTH / (v - 1)
    return pl.pallas_call(
        functools.partial(_tc_body, ni, eps, v),
        grid=(ni,),
        in_specs=[
            pl.BlockSpec((block_r, v), lambda i: (i + off_blocks, 0)),
            pl.BlockSpec((block_r, 1), lambda i: (i + off_blocks, 0)),
            pl.BlockSpec((block_r, 1), lambda i: (i + off_blocks, 0)),
        ],
        out_specs=pl.BlockSpec((1, 2), lambda i: (0, 0),
                               memory_space=pltpu.SMEM),
        out_shape=jax.ShapeDtypeStruct((1, 2), jnp.float32),
        scratch_shapes=[
            pltpu.SMEM((1, 1), jnp.float32),
            pltpu.SMEM((1, 1), jnp.float32),
        ],
    )


def kernel(input, target, mask):
    b, t, v = input.shape
    n = b * t
    x = input.reshape(n, v)
    tgt = target.reshape(n).astype(jnp.int32)
    m = mask.reshape(n)

    eps = SMOOTH / (v - 1)
    c_const = (v - 1) * eps * math.log(eps) + CONF * math.log(CONF)

    sc_part = _make_sc(n, v)(x, tgt, m)
    tc_out = _make_tc(n, v, 128, _ROWS_SC)(
        x, m.reshape(n, 1), tgt.reshape(n, 1))

    sc3 = sc_part.reshape(_NW, 3, _LANES)
    ts_sc = jnp.sum(sc3[:, 0, :])
    g_sc = jnp.sum(sc3[:, 1, :])
    mt = tc_out[0, 1] + jnp.sum(sc3[:, 2, :])
    total = tc_out[0, 0] - eps * ts_sc + (eps - CONF) * g_sc
    return (c_const * mt + total) / mt


# speedup vs baseline: 1.0488x; 1.0488x over previous
import functools
import math

import jax
import jax.numpy as jnp
from jax import lax
from jax.experimental import pallas as pl
from jax.experimental.pallas import tpu as pltpu
from jax.experimental.pallas import tpu_sc as plsc

SMOOTH = 0.1
CONF = 1.0 - SMOOTH

_NC = 2
_NS = 16
_NW = _NC * _NS
_LANES = 16


def _sc_body(n_rows, v, x_hbm, tgt_hbm, m_hbm, out_hbm,
             tgt_v, m_v, gbuf, acc_out, sem):
    per_w = n_rows // _NW
    wid = lax.axis_index("s") * _NC + lax.axis_index("c")
    rbase = wid * per_w

    pltpu.sync_copy(tgt_hbm.at[pl.ds(rbase, per_w)], tgt_v)
    pltpu.sync_copy(m_hbm.at[pl.ds(rbase, per_w)], m_v)

    nvec = per_w // _LANES
    tregs = [tgt_v[pl.ds(c * _LANES, _LANES)] for c in range(nvec)]

    handles = []
    for r in range(per_w):
        t_r = tregs[r // _LANES][r % _LANES]
        off = pl.multiple_of(t_r & ~(_LANES - 1), _LANES)
        handles.append(pltpu.async_copy(
            x_hbm.at[rbase + r, pl.ds(off, _LANES)],
            gbuf.at[pl.ds(r * _LANES, _LANES)], sem))
    for h in handles:
        h.wait()

    lane = lax.broadcasted_iota(jnp.int32, (_LANES,), 0)
    g16 = jnp.zeros((_LANES,), jnp.float32)
    ms16 = jnp.zeros((_LANES,), jnp.float32)
    for c in range(nvec):
        ms16 = ms16 + m_v[pl.ds(c * _LANES, _LANES)]
    mregs = [m_v[pl.ds(c * _LANES, _LANES)] for c in range(nvec)]
    for r in range(per_w):
        t_r = tregs[r // _LANES][r % _LANES]
        m_r = mregs[r // _LANES][r % _LANES]
        gv16 = gbuf[pl.ds(r * _LANES, _LANES)]
        eq = lane == (t_r & (_LANES - 1))
        g16 = g16 + jnp.where(eq, gv16 * m_r, 0.0)

    acc_out[pl.ds(0, _LANES)] = g16
    acc_out[pl.ds(_LANES, _LANES)] = ms16
    pltpu.sync_copy(acc_out, out_hbm.at[wid])


def _make_sc(n_rows, v):
    per_w = n_rows // _NW
    mesh = plsc.VectorSubcoreMesh(core_axis_name="c", subcore_axis_name="s")
    return pl.kernel(
        functools.partial(_sc_body, n_rows, v),
        out_type=jax.ShapeDtypeStruct((_NW, 2 * _LANES), jnp.float32),
        mesh=mesh,
        scratch_types=[
            pltpu.VMEM((per_w,), jnp.int32),
            pltpu.VMEM((per_w,), jnp.float32),
            pltpu.VMEM((per_w * _LANES,), jnp.float32),
            pltpu.VMEM((2 * _LANES,), jnp.float32),
            pltpu.SemaphoreType.DMA,
        ],
        compiler_params=pltpu.CompilerParams(use_tc_tiling_on_sc=True),
    )


def _tc_body(x_ref, m_ref, out_ref, acc_s):
    i = pl.program_id(0)

    @pl.when(i == 0)
    def _init():
        acc_s[0, 0] = 0.0

    x = x_ref[...]
    m = m_ref[...]
    rowsum = jnp.sum(x, axis=1, keepdims=True)
    acc_s[0, 0] += jnp.sum(rowsum * m)

    @pl.when(i == pl.num_programs(0) - 1)
    def _fin():
        out_ref[0, 0] = acc_s[0, 0]


def _make_tc(n_rows, v, block_r):
    ni = n_rows // block_r
    return pl.pallas_call(
        _tc_body,
        grid=(ni,),
        in_specs=[
            pl.BlockSpec((block_r, v), lambda i: (i, 0)),
            pl.BlockSpec((block_r, 1), lambda i: (i, 0)),
        ],
        out_specs=pl.BlockSpec((1, 1), lambda i: (0, 0),
                               memory_space=pltpu.SMEM),
        out_shape=jax.ShapeDtypeStruct((1, 1), jnp.float32),
        scratch_shapes=[
            pltpu.SMEM((1, 1), jnp.float32),
        ],
    )


def kernel(input, target, mask):
    b, t, v = input.shape
    n = b * t
    x = input.reshape(n, v)
    tgt = target.reshape(n).astype(jnp.int32)
    m = mask.reshape(n)

    eps = SMOOTH / (v - 1)
    c_const = (v - 1) * eps * math.log(eps) + CONF * math.log(CONF)

    sc_part = _make_sc(n, v)(x, tgt, m)
    ts = _make_tc(n, v, 128)(x, m.reshape(n, 1))[0, 0]

    sc2 = sc_part.reshape(_NW, 2, _LANES)
    g_tot = jnp.sum(sc2[:, 0, :])
    mt = jnp.sum(sc2[:, 1, :])
    return (c_const * mt - eps * ts + (eps - CONF) * g_tot) / mt


# speedup vs baseline: 1.0608x; 1.0115x over previous
import functools
import math

import jax
import jax.numpy as jnp
from jax import lax
from jax.experimental import pallas as pl
from jax.experimental.pallas import tpu as pltpu
from jax.experimental.pallas import tpu_sc as plsc

SMOOTH = 0.1
CONF = 1.0 - SMOOTH

_NC = 2
_NS = 16
_NW = _NC * _NS
_LANES = 16


def _sc_body(n_rows, v, x_hbm, tgt_hbm, m_hbm, out_hbm,
             tgt_v, m_v, gbuf, acc_out, sem):
    per_w = n_rows // _NW
    wid = lax.axis_index("s") * _NC + lax.axis_index("c")
    rbase = wid * per_w

    pltpu.sync_copy(tgt_hbm.at[pl.ds(rbase, per_w)], tgt_v)
    pltpu.sync_copy(m_hbm.at[pl.ds(rbase, per_w)], m_v)

    nvec = per_w // _LANES
    tregs = [tgt_v[pl.ds(c * _LANES, _LANES)] for c in range(nvec)]

    handles = []
    for r in range(per_w):
        t_r = tregs[r // _LANES][r % _LANES]
        off = pl.multiple_of(t_r & ~(_LANES - 1), _LANES)
        handles.append(pltpu.async_copy(
            x_hbm.at[rbase + r, pl.ds(off, _LANES)],
            gbuf.at[pl.ds(r * _LANES, _LANES)], sem))
    for h in handles:
        h.wait()

    lane = lax.broadcasted_iota(jnp.int32, (_LANES,), 0)
    g16 = jnp.zeros((_LANES,), jnp.float32)
    ms16 = jnp.zeros((_LANES,), jnp.float32)
    for c in range(nvec):
        ms16 = ms16 + m_v[pl.ds(c * _LANES, _LANES)]
    mregs = [m_v[pl.ds(c * _LANES, _LANES)] for c in range(nvec)]
    for r in range(per_w):
        t_r = tregs[r // _LANES][r % _LANES]
        m_r = mregs[r // _LANES][r % _LANES]
        gv16 = gbuf[pl.ds(r * _LANES, _LANES)]
        eq = lane == (t_r & (_LANES - 1))
        g16 = g16 + jnp.where(eq, gv16 * m_r, 0.0)

    acc_out[pl.ds(0, _LANES)] = g16
    acc_out[pl.ds(_LANES, _LANES)] = ms16
    pltpu.sync_copy(acc_out, out_hbm.at[wid])


def _make_sc(n_rows, v):
    per_w = n_rows // _NW
    mesh = plsc.VectorSubcoreMesh(core_axis_name="c", subcore_axis_name="s")
    return pl.kernel(
        functools.partial(_sc_body, n_rows, v),
        out_type=jax.ShapeDtypeStruct((_NW, 2 * _LANES), jnp.float32),
        mesh=mesh,
        scratch_types=[
            pltpu.VMEM((per_w,), jnp.int32),
            pltpu.VMEM((per_w,), jnp.float32),
            pltpu.VMEM((per_w * _LANES,), jnp.float32),
            pltpu.VMEM((2 * _LANES,), jnp.float32),
            pltpu.SemaphoreType.DMA,
        ],
        compiler_params=pltpu.CompilerParams(use_tc_tiling_on_sc=True),
    )


def _tc_body(c_const, eps, x_ref, m_ref, sc_ref, out_ref, acc_s):
    i = pl.program_id(0)

    @pl.when(i == 0)
    def _init():
        acc_s[0, 0] = 0.0

    x = x_ref[...]
    m = m_ref[...]
    rowsum = jnp.sum(x, axis=1, keepdims=True)
    acc_s[0, 0] += jnp.sum(rowsum * m)

    @pl.when(i == pl.num_programs(0) - 1)
    def _fin():
        sc = sc_ref[...]
        g_tot = jnp.sum(sc[:, :_LANES])
        mt = jnp.sum(sc[:, _LANES:])
        out_ref[0, 0] = (c_const * mt - eps * acc_s[0, 0]
                         + (eps - CONF) * g_tot) / mt


def _make_tc(n_rows, v, block_r):
    ni = n_rows // block_r
    eps = SMOOTH / (v - 1)
    c_const = (v - 1) * eps * math.log(eps) + CONF * math.log(CONF)
    return pl.pallas_call(
        functools.partial(_tc_body, c_const, eps),
        grid=(ni,),
        in_specs=[
            pl.BlockSpec((block_r, v), lambda i: (i, 0)),
            pl.BlockSpec((block_r, 1), lambda i: (i, 0)),
            pl.BlockSpec((_NW, 2 * _LANES), lambda i: (0, 0)),
        ],
        out_specs=pl.BlockSpec((1, 1), lambda i: (0, 0),
                               memory_space=pltpu.SMEM),
        out_shape=jax.ShapeDtypeStruct((1, 1), jnp.float32),
        scratch_shapes=[
            pltpu.SMEM((1, 1), jnp.float32),
        ],
    )


def kernel(input, target, mask):
    b, t, v = input.shape
    n = b * t
    x = input.reshape(n, v)
    tgt = target.reshape(n).astype(jnp.int32)
    m = mask.reshape(n)

    sc_part = _make_sc(n, v)(x, tgt, m)
    return _make_tc(n, v, 128)(x, m.reshape(n, 1), sc_part)[0, 0]
